# baseline (speedup 1.0000x reference)
import functools

import jax
import jax.numpy as jnp
from jax import lax
from jax.experimental import pallas as pl
from jax.experimental.pallas import tpu as pltpu
from jax.experimental.pallas import tpu_sc as plsc

NUM_NODES = 100000
NODE_DIM = 128
GLOBAL_DIM = 128
BATCH_SIZE = 256
MAX_CUBES = 256
MAX_MOVES = 24
HIDDEN = 128
NEG = -1000000000.0

NSLOTS = BATCH_SIZE * MAX_CUBES
NC, NS = 2, 16
NW = NC * NS
SLOTS_PER_W = NSLOTS // NW
GCHUNK = 128

@functools.cache
def _make_sc_gather():
    mesh = plsc.VectorSubcoreMesh(
        core_axis_name="c", subcore_axis_name="s",
        num_cores=NC, num_subcores=NS)

    @functools.partial(
        pl.kernel,
        out_type=jax.ShapeDtypeStruct((NSLOTS, NODE_DIM), jnp.float32),
        mesh=mesh,
        scratch_types=[
            pltpu.VMEM((GCHUNK,), jnp.int32),
            pltpu.VMEM((GCHUNK, NODE_DIM), jnp.float32),
            pltpu.SemaphoreType.DMA,
        ],
    )
    def sc_gather(table_hbm, idx_hbm, out_hbm, idx_v, rows_v, sem):
        wid = lax.axis_index("s") * NC + lax.axis_index("c")
        base = wid * SLOTS_PER_W
        for ch in range(SLOTS_PER_W // GCHUNK):
            off = base + ch * GCHUNK
            pltpu.sync_copy(idx_hbm.at[pl.ds(off, GCHUNK)], idx_v)
            pltpu.async_copy(table_hbm.at[idx_v], rows_v, sem).wait()
            pltpu.sync_copy(rows_v, out_hbm.at[pl.ds(off, GCHUNK)])

    return sc_gather


def _sc_gather(table, idx):
    return _make_sc_gather()(table, idx)


TB = 8


def _mlp_body(xg_ref, gf_ref, cnt_ref, w1n_ref, w1g_ref, b1_ref, w2_ref,
              b2_ref, mm_ref, out_ref):
    gf = gf_ref[...]
    gv = lax.dot_general(gf, w1g_ref[...], (((1,), (1,)), ((), ())),
                         preferred_element_type=jnp.float32) + b1_ref[...]
    x = xg_ref[...]
    h = lax.dot_general(x, w1n_ref[...], (((1,), (1,)), ((), ())),
                        preferred_element_type=jnp.float32)
    gvx = jnp.broadcast_to(gv.reshape(TB, 1, HIDDEN),
                           (TB, MAX_CUBES, HIDDEN)).reshape(TB * MAX_CUBES,
                                                            HIDDEN)
    h = jnp.maximum(h + gvx, 0.0)
    s = lax.dot_general(h, w2_ref[...], (((1,), (1,)), ((), ())),
                        preferred_element_type=jnp.float32) + b2_ref[...]
    s3 = s.reshape(TB, MAX_CUBES, MAX_MOVES)
    cnt = cnt_ref[0, 0, :].reshape(TB, 1, 1)
    c_iota = lax.broadcasted_iota(jnp.int32, (TB, MAX_CUBES, 1), 1)
    ok = (c_iota < jnp.minimum(cnt, MAX_CUBES)) & mm_ref[...]
    out_ref[...] = jnp.where(ok, s3, NEG)


def _tc_mlp(xg, gf, counts3, w1n, w1g, b1, w2, b2, mm):
    return pl.pallas_call(
        _mlp_body,
        grid=(BATCH_SIZE // TB,),
        in_specs=[
            pl.BlockSpec((TB * MAX_CUBES, NODE_DIM), lambda i: (i, 0)),
            pl.BlockSpec((TB, GLOBAL_DIM), lambda i: (i, 0)),
            pl.BlockSpec((1, 1, TB), lambda i: (i, 0, 0)),
            pl.BlockSpec((HIDDEN, NODE_DIM), lambda i: (0, 0)),
            pl.BlockSpec((HIDDEN, GLOBAL_DIM), lambda i: (0, 0)),
            pl.BlockSpec((1, HIDDEN), lambda i: (0, 0)),
            pl.BlockSpec((MAX_MOVES, HIDDEN), lambda i: (0, 0)),
            pl.BlockSpec((1, MAX_MOVES), lambda i: (0, 0)),
            pl.BlockSpec((TB, MAX_CUBES, MAX_MOVES), lambda i: (i, 0, 0)),
        ],
        out_specs=pl.BlockSpec((TB, MAX_CUBES, MAX_MOVES), lambda i: (i, 0, 0)),
        out_shape=jax.ShapeDtypeStruct((BATCH_SIZE, MAX_CUBES, MAX_MOVES),
                                       jnp.float32),
    )(xg, gf, counts3, w1n, w1g, b1, w2, b2, mm)


def kernel(node_features, global_features, cube_mask, batch, move_mask, W1, b1,
           W2, b2):
    m = cube_mask.astype(jnp.int32)
    counts = jax.ops.segment_sum(m, batch, num_segments=BATCH_SIZE)
    offsets = jnp.cumsum(counts) - counts
    pos = (jnp.cumsum(m) - m) - offsets[batch]
    valid = cube_mask & (pos < MAX_CUBES)
    dst = jnp.where(valid, batch * MAX_CUBES + pos, jnp.int32(1 << 30))
    src_slots = jnp.zeros((NSLOTS,), jnp.int32).at[dst].set(
        jnp.arange(NUM_NODES, dtype=jnp.int32), mode='drop')
    xg = _sc_gather(node_features, src_slots)
    out3 = _tc_mlp(xg, global_features,
                   counts.reshape(BATCH_SIZE // TB, 1, TB).astype(jnp.int32),
                   W1[:, :NODE_DIM], W1[:, NODE_DIM:], b1.reshape(1, HIDDEN),
                   W2, b2.reshape(1, MAX_MOVES), move_mask)
    return out3.reshape(BATCH_SIZE, BATCH_SIZE * MAX_MOVES)


# speedup vs baseline: 1.4502x; 1.4502x over previous
import functools

import jax
import jax.numpy as jnp
from jax import lax
from jax.experimental import pallas as pl
from jax.experimental.pallas import tpu as pltpu
from jax.experimental.pallas import tpu_sc as plsc

NUM_NODES = 100000
NODE_DIM = 128
GLOBAL_DIM = 128
BATCH_SIZE = 256
MAX_CUBES = 256
MAX_MOVES = 24
HIDDEN = 128
NEG = -1000000000.0

NSLOTS = BATCH_SIZE * MAX_CUBES
NC, NS = 2, 16
NW = NC * NS
SLOTS_PER_W = NSLOTS // NW
GCHUNK = 128
NCHUNK = SLOTS_PER_W // GCHUNK

NROWS = 784
NPAD = NROWS * 128


def _idx_body(m_ref, bt_ref, dst_ref):
    m = m_ref[...]
    bt = bt_ref[...]
    mf = m.astype(jnp.float32)

    x = mf
    for k in (1, 2, 4, 8, 16, 32, 64):
        x = x + jnp.concatenate(
            [jnp.zeros((NROWS, k), jnp.float32), x[:, :-k]], axis=1)
    row_tot = x[:, 127:128]
    y = row_tot
    for k in (1, 2, 4, 8, 16, 32, 64, 128, 256, 512):
        y = y + jnp.concatenate(
            [jnp.zeros((k, 1), jnp.float32), y[:-k]], axis=0)
    incl = x + (y - row_tot)
    excl = incl - mf

    prev_col0 = jnp.concatenate(
        [jnp.full((1, 1), -1, jnp.int32), bt[:-1, 127:128]], axis=0)
    prev_b = jnp.concatenate([prev_col0, bt[:, :-1]], axis=1)
    sv = jnp.where(bt != prev_b, excl, -1.0)
    a = sv
    for k in (1, 2, 4, 8, 16, 32, 64):
        a = jnp.maximum(a, jnp.concatenate(
            [jnp.full((NROWS, k), -1.0, jnp.float32), a[:, :-k]], axis=1))
    rmax = a[:, 127:128]
    z = rmax
    for k in (1, 2, 4, 8, 16, 32, 64, 128, 256, 512):
        z = jnp.maximum(z, jnp.concatenate(
            [jnp.full((k, 1), -1.0, jnp.float32), z[:-k]], axis=0))
    prev_rmax = jnp.concatenate(
        [jnp.full((1, 1), -1.0, jnp.float32), z[:-1]], axis=0)
    seg = jnp.maximum(a, prev_rmax)

    pos = (excl - seg).astype(jnp.int32)
    ok = (m > 0) & (pos < MAX_CUBES)
    dst_ref[...] = jnp.where(ok, bt * MAX_CUBES + pos, jnp.int32(1 << 30))


def _tc_idx(m2d, bt2d):
    return pl.pallas_call(
        _idx_body,
        in_specs=[
            pl.BlockSpec((NROWS, 128), lambda: (0, 0)),
            pl.BlockSpec((NROWS, 128), lambda: (0, 0)),
        ],
        out_specs=pl.BlockSpec((NROWS, 128), lambda: (0, 0)),
        out_shape=jax.ShapeDtypeStruct((NROWS, 128), jnp.int32),
    )(m2d, bt2d)


@functools.cache
def _make_sc_gather():
    mesh = plsc.VectorSubcoreMesh(
        core_axis_name="c", subcore_axis_name="s",
        num_cores=NC, num_subcores=NS)

    @functools.partial(
        pl.kernel,
        out_type=jax.ShapeDtypeStruct((NSLOTS, NODE_DIM), jnp.float32),
        mesh=mesh,
        scratch_types=[
            pltpu.VMEM((SLOTS_PER_W,), jnp.int32),
            pltpu.VMEM((GCHUNK, NODE_DIM), jnp.float32),
            pltpu.VMEM((GCHUNK, NODE_DIM), jnp.float32),
            pltpu.SemaphoreType.DMA,
            pltpu.SemaphoreType.DMA,
            pltpu.SemaphoreType.DMA,
            pltpu.SemaphoreType.DMA,
        ],
    )
    def sc_gather(table_hbm, idx_hbm, out_hbm, idx_all, rows_a, rows_b,
                  gsem_a, gsem_b, wsem_a, wsem_b):
        wid = lax.axis_index("s") * NC + lax.axis_index("c")
        base = wid * SLOTS_PER_W
        pltpu.sync_copy(idx_hbm.at[pl.ds(base, SLOTS_PER_W)], idx_all)
        bufs = [rows_a, rows_b]
        gs = [gsem_a, gsem_b]
        ws = [wsem_a, wsem_b]
        g = [None, None]
        w = [None, None]

        def start_gather(ch, p):
            g[p] = pltpu.make_async_copy(
                table_hbm.at[idx_all.at[pl.ds(ch * GCHUNK, GCHUNK)]],
                bufs[p], gs[p])
            g[p].start()

        start_gather(0, 0)
        for ch in range(NCHUNK):
            p = ch & 1
            q = p ^ 1
            g[p].wait()
            if ch + 1 < NCHUNK:
                if w[q] is not None:
                    w[q].wait()
                start_gather(ch + 1, q)
            w[p] = pltpu.make_async_copy(
                bufs[p], out_hbm.at[pl.ds(base + ch * GCHUNK, GCHUNK)], ws[p])
            w[p].start()
        w[(NCHUNK - 2) & 1].wait()
        w[(NCHUNK - 1) & 1].wait()

    return sc_gather


def _sc_gather(table, idx):
    return _make_sc_gather()(table, idx)


TB = 8


def _mlp_body(xg_ref, gf_ref, vld_ref, w1n_ref, w1g_ref, b1_ref, w2_ref,
              b2_ref, mm_ref, out_ref):
    gf = gf_ref[...]
    gv = lax.dot_general(gf, w1g_ref[...], (((1,), (1,)), ((), ())),
                         preferred_element_type=jnp.float32) + b1_ref[...]
    x = xg_ref[...]
    h = lax.dot_general(x, w1n_ref[...], (((1,), (1,)), ((), ())),
                        preferred_element_type=jnp.float32)
    gvx = jnp.broadcast_to(gv.reshape(TB, 1, HIDDEN),
                           (TB, MAX_CUBES, HIDDEN)).reshape(TB * MAX_CUBES,
                                                            HIDDEN)
    h = jnp.maximum(h + gvx, 0.0)
    s = lax.dot_general(h, w2_ref[...], (((1,), (1,)), ((), ())),
                        preferred_element_type=jnp.float32) + b2_ref[...]
    s3 = s.reshape(TB, MAX_CUBES, MAX_MOVES)
    ok = (vld_ref[...] > 0) & mm_ref[...]
    out_ref[...] = jnp.where(ok, s3, NEG)


def _tc_mlp(xg, gf, vld3, w1n, w1g, b1, w2, b2, mm):
    return pl.pallas_call(
        _mlp_body,
        grid=(BATCH_SIZE // TB,),
        in_specs=[
            pl.BlockSpec((TB * MAX_CUBES, NODE_DIM), lambda i: (i, 0)),
            pl.BlockSpec((TB, GLOBAL_DIM), lambda i: (i, 0)),
            pl.BlockSpec((TB, MAX_CUBES, 1), lambda i: (i, 0, 0)),
            pl.BlockSpec((HIDDEN, NODE_DIM), lambda i: (0, 0)),
            pl.BlockSpec((HIDDEN, GLOBAL_DIM), lambda i: (0, 0)),
            pl.BlockSpec((1, HIDDEN), lambda i: (0, 0)),
            pl.BlockSpec((MAX_MOVES, HIDDEN), lambda i: (0, 0)),
            pl.BlockSpec((1, MAX_MOVES), lambda i: (0, 0)),
            pl.BlockSpec((TB, MAX_CUBES, MAX_MOVES), lambda i: (i, 0, 0)),
        ],
        out_specs=pl.BlockSpec((TB, MAX_CUBES, MAX_MOVES), lambda i: (i, 0, 0)),
        out_shape=jax.ShapeDtypeStruct((BATCH_SIZE, MAX_CUBES, MAX_MOVES),
                                       jnp.float32),
    )(xg, gf, vld3, w1n, w1g, b1, w2, b2, mm)


def kernel(node_features, global_features, cube_mask, batch, move_mask, W1, b1,
           W2, b2):
    m2d = jnp.pad(cube_mask.astype(jnp.int32),
                  (0, NPAD - NUM_NODES)).reshape(NROWS, 128)
    bt2d = jnp.pad(batch.astype(jnp.int32), (0, NPAD - NUM_NODES),
                   constant_values=BATCH_SIZE - 1).reshape(NROWS, 128)
    dst = _tc_idx(m2d, bt2d).reshape(-1)[:NUM_NODES]
    src1 = jnp.zeros((NSLOTS,), jnp.int32).at[dst].set(
        jnp.arange(1, NUM_NODES + 1, dtype=jnp.int32), mode='drop')
    src_g = jnp.maximum(src1 - 1, 0)
    xg = _sc_gather(node_features, src_g)
    vld3 = src1.reshape(BATCH_SIZE, MAX_CUBES, 1)
    out3 = _tc_mlp(xg, global_features, vld3,
                   W1[:, :NODE_DIM], W1[:, NODE_DIM:], b1.reshape(1, HIDDEN),
                   W2, b2.reshape(1, MAX_MOVES), move_mask)
    return out3.reshape(BATCH_SIZE, BATCH_SIZE * MAX_MOVES)
